# baseline (speedup 1.0000x reference)
import functools
import math

import jax
import jax.numpy as jnp
from jax import lax
from jax.experimental import pallas as pl
from jax.experimental.pallas import tpu as pltpu
from jax.experimental.pallas import tpu_sc as plsc

_CHUNK = 512


@functools.lru_cache(maxsize=None)
def _build(B, V, D, L, NC, NS):
    NW = NC * NS
    b_per_w = B // NW
    n_chunks = b_per_w // _CHUNK
    scale = math.sqrt(D)
    mesh = plsc.VectorSubcoreMesh(core_axis_name="c", subcore_axis_name="s")

    @functools.partial(
        pl.kernel,
        mesh=mesh,
        compiler_params=pltpu.CompilerParams(use_tc_tiling_on_sc=False),
        out_type=jax.ShapeDtypeStruct((B, D), jnp.float32),
        scratch_types=[
            pltpu.VMEM((_CHUNK,), jnp.int32),
            pltpu.VMEM((_CHUNK, D), jnp.float32),
            pltpu.SemaphoreType.DMA,
        ],
    )
    def k(idx_hbm, table_hbm, out_hbm, idx_v, rows_v, sem):
        wid = lax.axis_index("s") * NC + lax.axis_index("c")
        base = wid * b_per_w

        def chunk_body(g, carry):
            off = base + g * _CHUNK
            pltpu.sync_copy(idx_hbm.at[pl.ds(off, _CHUNK)], idx_v)
            pltpu.async_copy(table_hbm.at[idx_v], rows_v, sem).wait()

            def scale_body(i, c):
                for j in range(D // L):
                    s = pl.ds(j * L, L)
                    rows_v[i, s] = rows_v[i, s] * scale
                return c

            lax.fori_loop(0, _CHUNK, scale_body, 0)
            pltpu.sync_copy(rows_v, out_hbm.at[pl.ds(off, _CHUNK)])
            return carry

        lax.fori_loop(0, n_chunks, chunk_body, 0)

    return k


def kernel(x, table):
    R, S = x.shape
    V, D = table.shape
    B = R * S
    info = plsc.get_sparse_core_info()
    k = _build(B, V, D, info.num_lanes, info.num_cores, info.num_subcores)
    out = k(x.reshape(B), table)
    return out.reshape(R, S, D)


# speedup vs baseline: 1.1351x; 1.1351x over previous
import functools
import math

import jax
import jax.numpy as jnp
from jax import lax
from jax.experimental import pallas as pl
from jax.experimental.pallas import tpu as pltpu
from jax.experimental.pallas import tpu_sc as plsc

_CHUNK = 400


@functools.lru_cache(maxsize=None)
def _build(B, V, D, L, NC, NS):
    NW = NC * NS
    b_per_w = B // NW
    C = _CHUNK
    n_chunks = b_per_w // C
    assert n_chunks % 2 == 0 and n_chunks >= 6
    scale = math.sqrt(D)
    mesh = plsc.VectorSubcoreMesh(core_axis_name="c", subcore_axis_name="s")

    @functools.partial(
        pl.kernel,
        mesh=mesh,
        compiler_params=pltpu.CompilerParams(use_tc_tiling_on_sc=False),
        out_type=jax.ShapeDtypeStruct((B, D), jnp.float32),
        scratch_types=[
            pltpu.VMEM((2, C), jnp.int32),
            pltpu.VMEM((C, D), jnp.float32),
            pltpu.VMEM((C, D), jnp.float32),
            pltpu.VMEM((C, D), jnp.float32),
            pltpu.VMEM((C, D), jnp.float32),
            pltpu.SemaphoreType.DMA,
            pltpu.SemaphoreType.DMA,
            pltpu.SemaphoreType.DMA,
            pltpu.SemaphoreType.DMA,
        ],
    )
    def k(idx_hbm, table_hbm, out_hbm, idx_v, g0, g1, w0, w1,
          sin0, sin1, sout0, sout1):
        gbuf = (g0, g1)
        wbuf = (w0, w1)
        sin = (sin0, sin1)
        sout = (sout0, sout1)
        wid = lax.axis_index("s") * NC + lax.axis_index("c")
        base = wid * b_per_w

        def issue_gather(g, b):
            off = base + g * C
            pltpu.sync_copy(idx_hbm.at[pl.ds(off, C)], idx_v.at[b])
            pltpu.async_copy(table_hbm.at[idx_v.at[b]], gbuf[b], sin[b])

        def wait_gather(b):
            pltpu.make_async_copy(table_hbm.at[idx_v.at[b]], gbuf[b],
                                  sin[b]).wait()

        def scale_chunk(b):
            src, dst = gbuf[b], wbuf[b]

            def body(i, c):
                for r in range(2):
                    for j in range(D // L):
                        s = pl.ds(j * L, L)
                        dst[2 * i + r, s] = src[2 * i + r, s] * scale
                return c

            lax.fori_loop(0, C // 2, body, 0)

        def issue_write(g, b):
            off = base + g * C
            pltpu.async_copy(wbuf[b], out_hbm.at[pl.ds(off, C)], sout[b])

        def wait_write(g, b):
            off = base + g * C
            pltpu.make_async_copy(wbuf[b], out_hbm.at[pl.ds(off, C)],
                                  sout[b]).wait()

        for g in range(2):
            issue_gather(g, g)
        for g in range(2):
            b = g % 2
            wait_gather(b)
            scale_chunk(b)
            issue_write(g, b)
            issue_gather(g + 2, b)

        def outer(gg, c):
            for b in range(2):
                g = 2 + 2 * gg + b
                wait_gather(b)
                wait_write(g - 2, b)
                scale_chunk(b)
                issue_write(g, b)
                issue_gather(g + 2, b)
            return c

        lax.fori_loop(0, (n_chunks - 4) // 2, outer, 0)

        for g in range(n_chunks - 2, n_chunks):
            b = g % 2
            wait_gather(b)
            wait_write(g - 2, b)
            scale_chunk(b)
            issue_write(g, b)
        for g in range(n_chunks - 2, n_chunks):
            wait_write(g, g % 2)

    return k


def kernel(x, table):
    R, S = x.shape
    V, D = table.shape
    B = R * S
    info = plsc.get_sparse_core_info()
    k = _build(B, V, D, info.num_lanes, info.num_cores, info.num_subcores)
    out = k(x.reshape(B), table)
    return out.reshape(R, S, D)


# speedup vs baseline: 1.1361x; 1.0008x over previous
import functools
import math

import jax
import jax.numpy as jnp
from jax import lax
from jax.experimental import pallas as pl
from jax.experimental.pallas import tpu as pltpu
from jax.experimental.pallas import tpu_sc as plsc

_RO = 8


@functools.lru_cache(maxsize=None)
def _build(R, S, V, D, L, NC, NS):
    NW = NC * NS
    C = _RO * S
    r_per_w = R // NW
    n_chunks = r_per_w // _RO
    assert n_chunks % 2 == 0 and n_chunks >= 6
    scale = math.sqrt(D)
    mesh = plsc.VectorSubcoreMesh(core_axis_name="c", subcore_axis_name="s")

    @functools.partial(
        pl.kernel,
        mesh=mesh,
        compiler_params=pltpu.CompilerParams(use_tc_tiling_on_sc=False),
        out_type=jax.ShapeDtypeStruct((R, S, D), jnp.float32),
        scratch_types=[
            pltpu.VMEM((2, C), jnp.int32),
            pltpu.VMEM((C, D), jnp.float32),
            pltpu.VMEM((C, D), jnp.float32),
            pltpu.VMEM((_RO, S, D), jnp.float32),
            pltpu.VMEM((_RO, S, D), jnp.float32),
            pltpu.SemaphoreType.DMA,
            pltpu.SemaphoreType.DMA,
            pltpu.SemaphoreType.DMA,
            pltpu.SemaphoreType.DMA,
        ],
    )
    def k(idx_hbm, table_hbm, out_hbm, idx_v, g0, g1, w0, w1,
          sin0, sin1, sout0, sout1):
        gbuf = (g0, g1)
        wbuf = (w0, w1)
        sin = (sin0, sin1)
        sout = (sout0, sout1)
        wid = lax.axis_index("s") * NC + lax.axis_index("c")
        base = wid * r_per_w

        def issue_gather(g, b):
            off = (base + g * _RO) * S
            pltpu.sync_copy(idx_hbm.at[pl.ds(off, C)], idx_v.at[b])
            pltpu.async_copy(table_hbm.at[idx_v.at[b]], gbuf[b], sin[b])

        def wait_gather(b):
            pltpu.make_async_copy(table_hbm.at[idx_v.at[b]], gbuf[b],
                                  sin[b]).wait()

        def scale_chunk(b):
            src, dst = gbuf[b], wbuf[b]

            def body(s, c):
                for r in range(_RO):
                    for j in range(D // L):
                        sl = pl.ds(j * L, L)
                        dst[r, s, sl] = src[r * S + s, sl] * scale
                return c

            lax.fori_loop(0, S, body, 0)

        def issue_write(g, b):
            off = base + g * _RO
            pltpu.async_copy(wbuf[b], out_hbm.at[pl.ds(off, _RO)], sout[b])

        def wait_write(g, b):
            off = base + g * _RO
            pltpu.make_async_copy(wbuf[b], out_hbm.at[pl.ds(off, _RO)],
                                  sout[b]).wait()

        for g in range(2):
            issue_gather(g, g)
        for g in range(2):
            b = g % 2
            wait_gather(b)
            scale_chunk(b)
            issue_write(g, b)
            issue_gather(g + 2, b)

        def outer(gg, c):
            for b in range(2):
                g = 2 + 2 * gg + b
                wait_gather(b)
                wait_write(g - 2, b)
                scale_chunk(b)
                issue_write(g, b)
                issue_gather(g + 2, b)
            return c

        lax.fori_loop(0, (n_chunks - 4) // 2, outer, 0)

        for g in range(n_chunks - 2, n_chunks):
            b = g % 2
            wait_gather(b)
            wait_write(g - 2, b)
            scale_chunk(b)
            issue_write(g, b)
        for g in range(n_chunks - 2, n_chunks):
            wait_write(g, g % 2)

    return k


def kernel(x, table):
    R, S = x.shape
    V, D = table.shape
    info = plsc.get_sparse_core_info()
    k = _build(R, S, V, D, info.num_lanes, info.num_cores, info.num_subcores)
    return k(x.reshape(R * S), table)


# speedup vs baseline: 1.1999x; 1.0562x over previous
import functools
import math

import jax
import jax.numpy as jnp
from jax import lax
from jax.experimental import pallas as pl
from jax.experimental.pallas import tpu as pltpu
from jax.experimental.pallas import tpu_sc as plsc

_G = 25
_RO = 4


@functools.lru_cache(maxsize=None)
def _build_stage(V, D, L, NC, NS):
    NW = NC * NS
    n_groups = V // 8
    n_chunks_total = n_groups // _G
    assert n_groups % _G == 0
    scale = math.sqrt(D)
    mesh = plsc.VectorSubcoreMesh(core_axis_name="c", subcore_axis_name="s")

    @functools.partial(
        pl.kernel,
        mesh=mesh,
        out_type=jax.ShapeDtypeStruct((n_groups, 8, 2 * D), jnp.float32),
        scratch_types=[
            pltpu.VMEM((_G, 8, D), jnp.float32),
            pltpu.VMEM((_G, 8, D), jnp.float32),
            pltpu.VMEM((_G, 8, 2 * D), jnp.float32),
            pltpu.VMEM((_G, 8, 2 * D), jnp.float32),
            pltpu.SemaphoreType.DMA,
            pltpu.SemaphoreType.DMA,
            pltpu.SemaphoreType.DMA,
            pltpu.SemaphoreType.DMA,
        ],
    )
    def k(tbl_hbm, pad_hbm, r0, r1, w0, w1, si0, si1, so0, so1):
        rbuf = (r0, r1)
        wbuf = (w0, w1)
        sin = (si0, si1)
        sout = (so0, so1)
        wid = lax.axis_index("s") * NC + lax.axis_index("c")
        n_mine = (n_chunks_total - wid + NW - 1) // NW

        def issue_read(t, b):
            g0 = (wid + t * NW) * _G
            pltpu.async_copy(tbl_hbm.at[pl.ds(g0, _G)], rbuf[b], sin[b])

        def wait_read(t, b):
            g0 = (wid + t * NW) * _G
            pltpu.make_async_copy(tbl_hbm.at[pl.ds(g0, _G)], rbuf[b],
                                  sin[b]).wait()

        def issue_write(t, b):
            g0 = (wid + t * NW) * _G
            pltpu.async_copy(wbuf[b], pad_hbm.at[pl.ds(g0, _G)], sout[b])

        def wait_write(t, b):
            g0 = (wid + t * NW) * _G
            pltpu.make_async_copy(wbuf[b], pad_hbm.at[pl.ds(g0, _G)],
                                  sout[b]).wait()

        def widen(b):
            src, dst = rbuf[b], wbuf[b]

            def body(g, c):
                for h in range(8):
                    for j in range(D // L):
                        sl = pl.ds(j * L, L)
                        dst[g, h, sl] = src[g, h, sl] * scale
                return c

            lax.fori_loop(0, _G, body, 0)

        @pl.when(n_mine > 0)
        def _():
            issue_read(0, 0)

            def body(t2, c):
                for b in range(2):
                    t = 2 * t2 + b

                    @pl.when(t < n_mine)
                    def _(t=t, b=b):
                        @pl.when(t + 1 < n_mine)
                        def _():
                            issue_read(t + 1, 1 - b)

                        wait_read(t, b)

                        @pl.when(t >= 2)
                        def _():
                            wait_write(t - 2, b)

                        widen(b)
                        issue_write(t, b)

                return c

            lax.fori_loop(0, (n_mine + 1) // 2, body, 0)
            for bb in range(2):
                t_last = n_mine - 1 - ((n_mine - 1 - bb) % 2)

                @pl.when(n_mine > bb)
                def _(bb=bb, t_last=t_last):
                    wait_write(t_last, bb)

    return k


@functools.lru_cache(maxsize=None)
def _build_gather(R, S, V, D, L, NC, NS):
    NW = NC * NS
    C = _RO * S
    r_per_w = R // NW
    n_chunks = r_per_w // _RO
    assert n_chunks % 2 == 0 and n_chunks >= 6
    mesh = plsc.VectorSubcoreMesh(core_axis_name="c", subcore_axis_name="s")

    @functools.partial(
        pl.kernel,
        mesh=mesh,
        out_type=jax.ShapeDtypeStruct((R, S, D), jnp.float32),
        scratch_types=[
            pltpu.VMEM((r_per_w * S,), jnp.int32),
            pltpu.VMEM((C, 2 * D), jnp.float32),
            pltpu.VMEM((C, 2 * D), jnp.float32),
            pltpu.VMEM((_RO, S, D), jnp.float32),
            pltpu.SemaphoreType.DMA,
            pltpu.SemaphoreType.DMA,
            pltpu.SemaphoreType.DMA,
        ],
    )
    def k(idx_hbm, pad_hbm, out_hbm, idx_v, g0, g1, w0,
          sin0, sin1, sout0):
        gbuf = (g0, g1)
        wbuf = w0
        sin = (sin0, sin1)
        sout = sout0
        wid = lax.axis_index("s") * NC + lax.axis_index("c")
        base = wid * r_per_w
        pltpu.sync_copy(idx_hbm.at[pl.ds(base * S, r_per_w * S)], idx_v)

        def issue_gather(g, b):
            pltpu.async_copy(pad_hbm.at[idx_v.at[pl.ds(g * C, C)]],
                             gbuf[b], sin[b])

        def wait_gather(g, b):
            pltpu.make_async_copy(pad_hbm.at[idx_v.at[pl.ds(g * C, C)]],
                                  gbuf[b], sin[b]).wait()

        def compact_chunk(b):
            src, dst = gbuf[b], wbuf

            def body(s, c):
                for r in range(_RO):
                    for j in range(D // L):
                        sl = pl.ds(j * L, L)
                        dst[r, s, sl] = src[r * S + s, sl]
                return c

            lax.fori_loop(0, S, body, 0)

        def issue_write(g):
            off = base + g * _RO
            pltpu.async_copy(wbuf, out_hbm.at[pl.ds(off, _RO)], sout)

        def wait_write(g):
            off = base + g * _RO
            pltpu.make_async_copy(wbuf, out_hbm.at[pl.ds(off, _RO)],
                                  sout).wait()

        for g in range(2):
            issue_gather(g, g)
        for g in range(2):
            b = g % 2
            wait_gather(g, b)
            if g > 0:
                wait_write(g - 1)
            compact_chunk(b)
            issue_write(g)
            issue_gather(g + 2, b)

        def outer(gg, c):
            for b in range(2):
                g = 2 + 2 * gg + b
                wait_gather(g, b)
                wait_write(g - 1)
                compact_chunk(b)
                issue_write(g)
                issue_gather(g + 2, b)
            return c

        lax.fori_loop(0, (n_chunks - 4) // 2, outer, 0)

        for g in range(n_chunks - 2, n_chunks):
            b = g % 2
            wait_gather(g, b)
            wait_write(g - 1)
            compact_chunk(b)
            issue_write(g)
        wait_write(n_chunks - 1)

    return k


def kernel(x, table):
    R, S = x.shape
    V, D = table.shape
    info = plsc.get_sparse_core_info()
    NC, NS, L = info.num_cores, info.num_subcores, info.num_lanes
    pad = _build_stage(V, D, L, NC, NS)(table.reshape(V // 8, 8, D))
    pad2 = pad.reshape(V, 2 * D)
    return _build_gather(R, S, V, D, L, NC, NS)(x.reshape(R * S), pad2)
